# Initial kernel scaffold; baseline (speedup 1.0000x reference)
import jax
import jax.numpy as jnp
from jax.experimental import pallas as pl


def kernel(x, edge_index, W0, a_src0, a_dst0, b0, W1, a_src1, a_dst1, b1):
    raise NotImplementedError("write your pallas kernel here")


# speedup vs baseline: 23.2762x; 23.2762x over previous
import functools

import jax
import jax.numpy as jnp
from jax import lax
from jax.experimental import pallas as pl
from jax.experimental.pallas import tpu as pltpu
from jax.experimental.pallas import tpu_sc as plsc

N = 10000
E = 320000
D = 128
NPAD = 10240
ROWS_PER_TILE = NPAD // 16
CH = 80
EDGES_PER_TILE = E // 32
NCHUNK = EDGES_PER_TILE // CH


def _head1_body(x_ref, w_ref, as_ref, ad_ref, h_ref, sas_ref, sad_ref):
    h = jnp.dot(x_ref[...], w_ref[...], preferred_element_type=jnp.float32)
    h_ref[...] = h
    sas_ref[...] = jnp.sum(h * as_ref[...], axis=1)
    sad_ref[...] = jnp.sum(h * ad_ref[...], axis=1)


def _mid_body(a0_ref, a1_ref, d0_ref, d1_ref, b_ref, w_ref, as_ref, ad_ref,
              h_ref, sas_ref, sad_ref):
    den = d0_ref[:N, :] + d1_ref[:N, :]
    rec = 1.0 / (den + 1e-16)
    s = a0_ref[:N, :] + a1_ref[:N, :]
    z = s * rec + b_ref[...]
    z = jnp.where(z > 0.0, z, jnp.exp(z) - 1.0)
    h = jnp.dot(z, w_ref[...], preferred_element_type=jnp.float32)
    h_ref[...] = h
    sas_ref[...] = jnp.sum(h * as_ref[...], axis=1)
    sad_ref[...] = jnp.sum(h * ad_ref[...], axis=1)


def _final_body(a0_ref, a1_ref, d0_ref, d1_ref, b_ref, o_ref):
    den = d0_ref[:N, :] + d1_ref[:N, :]
    rec = 1.0 / (den + 1e-16)
    s = a0_ref[:N, :] + a1_ref[:N, :]
    o_ref[...] = s * rec + b_ref[...]


_OUT_HEAD = [
    jax.ShapeDtypeStruct((N, D), jnp.float32),
    jax.ShapeDtypeStruct((N,), jnp.float32),
    jax.ShapeDtypeStruct((N,), jnp.float32),
]

_head1 = pl.pallas_call(_head1_body, out_shape=_OUT_HEAD)
_mid = pl.pallas_call(_mid_body, out_shape=_OUT_HEAD)
_final = pl.pallas_call(
    _final_body, out_shape=jax.ShapeDtypeStruct((N, D), jnp.float32)
)


def _vmax_all(ref):
    def body(i, m):
        return jnp.maximum(m, ref[pl.ds(i * 16, 16)])
    m = lax.fori_loop(0, N // 16, body, jnp.full((16,), -jnp.inf, jnp.float32))
    s = m[0]
    for i in range(1, 16):
        s = jnp.maximum(s, m[i])
    return s


@functools.lru_cache(maxsize=None)
def _make_sc_edge():
    mesh = plsc.VectorSubcoreMesh(
        core_axis_name="c", subcore_axis_name="s", num_cores=2, num_subcores=16
    )

    @functools.partial(
        pl.kernel,
        out_type=(
            jax.ShapeDtypeStruct((NPAD, D), jnp.float32),
            jax.ShapeDtypeStruct((NPAD, D), jnp.float32),
            jax.ShapeDtypeStruct((NPAD,), jnp.float32),
            jax.ShapeDtypeStruct((NPAD,), jnp.float32),
        ),
        mesh=mesh,
        compiler_params=pltpu.CompilerParams(needs_layout_passes=False),
        scratch_types=dict(
            asb=pltpu.VMEM((N,), jnp.float32),
            adb=pltpu.VMEM((N,), jnp.float32),
            srcv=pltpu.VMEM((CH,), jnp.int32),
            dstv=pltpu.VMEM((CH,), jnp.int32),
            exv=pltpu.VMEM((CH,), jnp.float32),
            rowbuf=pltpu.VMEM((CH, D), jnp.float32),
            zb1=pltpu.VMEM((ROWS_PER_TILE,), jnp.float32),
            acc_sh=pltpu.VMEM_SHARED((NPAD, D), jnp.float32),
            den_sh=pltpu.VMEM_SHARED((NPAD,), jnp.float32),
            sem=pltpu.SemaphoreType.DMA,
        ),
    )
    def sc_edge(h, asv, adv, src, dst, acc0, acc1, den0, den1,
                asb, adb, srcv, dstv, exv, rowbuf, zb1, acc_sh, den_sh, sem):
        cidx = lax.axis_index("c")
        sidx = lax.axis_index("s")

        pltpu.sync_copy(asv, asb)
        pltpu.sync_copy(adv, adb)

        zeros16 = jnp.zeros((16,), jnp.float32)

        def zrow(j, c):
            for k in range(D // 16):
                rowbuf[j, pl.ds(k * 16, 16)] = zeros16
            return c
        lax.fori_loop(0, CH, zrow, 0)

        def z1(j, c):
            zb1[pl.ds(j * 16, 16)] = zeros16
            return c
        lax.fori_loop(0, ROWS_PER_TILE // 16, z1, 0)

        r0 = sidx * ROWS_PER_TILE
        for jb in range(ROWS_PER_TILE // CH):
            rb = pl.multiple_of(r0 + jb * CH, 8)
            pltpu.sync_copy(rowbuf, acc_sh.at[pl.ds(rb, CH)])
        pltpu.sync_copy(zb1, den_sh.at[pl.ds(pl.multiple_of(r0, 8), ROWS_PER_TILE)])

        m_as = _vmax_all(asb)
        m_ad = _vmax_all(adb)
        msum = m_as + m_ad
        mbound = jnp.where(msum >= 0.0, msum, 0.2 * msum)

        plsc.subcore_barrier()

        ebase = (cidx * 16 + sidx) * EDGES_PER_TILE

        def chunk(i, c):
            base = pl.multiple_of(ebase + i * CH, 8)
            pltpu.sync_copy(src.at[pl.ds(base, CH)], srcv)
            pltpu.sync_copy(dst.at[pl.ds(base, CH)], dstv)
            for j in range(CH // 16):
                sv = srcv[pl.ds(j * 16, 16)]
                dv = dstv[pl.ds(j * 16, 16)]
                av = plsc.load_gather(asb, [sv])
                bv = plsc.load_gather(adb, [dv])
                e = av + bv
                e = jnp.where(e >= 0.0, e, 0.2 * e) - mbound
                exv[pl.ds(j * 16, 16)] = jnp.exp(e)

            pltpu.async_copy(h.at[srcv], rowbuf, sem).wait()

            def sblk(b, c2):
                ex16 = exv[pl.ds(b * 16, 16)]
                for jj in range(16):
                    j = b * 16 + jj
                    s_ = ex16[jj]
                    for k in range(D // 16):
                        rowbuf[j, pl.ds(k * 16, 16)] = (
                            rowbuf[j, pl.ds(k * 16, 16)] * s_
                        )
                return c2
            lax.fori_loop(0, CH // 16, sblk, 0)

            pltpu.sync_copy(exv, den_sh.at[dstv], add=True)
            pltpu.sync_copy(rowbuf, acc_sh.at[dstv], add=True)
            return c
        lax.fori_loop(0, NCHUNK, chunk, 0)

        plsc.subcore_barrier()

        rr = pl.multiple_of(r0, 8)

        @pl.when(cidx == 0)
        def _():
            pltpu.sync_copy(acc_sh.at[pl.ds(rr, ROWS_PER_TILE)],
                            acc0.at[pl.ds(rr, ROWS_PER_TILE)])
            pltpu.sync_copy(den_sh.at[pl.ds(rr, ROWS_PER_TILE)],
                            den0.at[pl.ds(rr, ROWS_PER_TILE)])

        @pl.when(cidx == 1)
        def _():
            pltpu.sync_copy(acc_sh.at[pl.ds(rr, ROWS_PER_TILE)],
                            acc1.at[pl.ds(rr, ROWS_PER_TILE)])
            pltpu.sync_copy(den_sh.at[pl.ds(rr, ROWS_PER_TILE)],
                            den1.at[pl.ds(rr, ROWS_PER_TILE)])

    return sc_edge


def kernel(x, edge_index, W0, a_src0, a_dst0, b0, W1, a_src1, a_dst1, b1):
    src = edge_index[0].astype(jnp.int32)
    dst = edge_index[1].astype(jnp.int32)
    sc_edge = _make_sc_edge()
    h0, as0, ad0 = _head1(x, W0, a_src0, a_dst0)
    a0, a1, d0, d1 = sc_edge(h0, as0, ad0, src, dst)
    h1, as1, ad1 = _mid(a0, a1, d0.reshape(NPAD, 1), d1.reshape(NPAD, 1),
                        b0, W1, a_src1, a_dst1)
    a0, a1, d0, d1 = sc_edge(h1, as1, ad1, src, dst)
    return _final(a0, a1, d0.reshape(NPAD, 1), d1.reshape(NPAD, 1), b1)
